# Initial kernel scaffold; baseline (speedup 1.0000x reference)
import jax
import jax.numpy as jnp
from jax.experimental import pallas as pl


def kernel(h, edge_index, W, b):
    raise NotImplementedError("write your pallas kernel here")


# speedup vs baseline: 8.1946x; 8.1946x over previous
import dataclasses
import functools

import jax
import jax.numpy as jnp
from jax import lax
from jax.experimental import pallas as pl
from jax.experimental.pallas import tpu as pltpu
from jax.experimental.pallas import tpu_sc as plsc

N_NODES = 10000
N_EDGES = 320000
DIM = 128
HALF = DIM // 2
NC = 2
NS = 16
NW = NC * NS
N_PAD = 10240
ROWS_PER_TILE = N_PAD // NS
CHUNK = 128
EPW = N_EDGES // NW
CHUNKS = 80
EPW_PAD = CHUNKS * CHUNK
PAD_E = EPW_PAD - EPW

_mesh = plsc.VectorSubcoreMesh(
    core_axis_name="c", subcore_axis_name="s", num_cores=NC, num_subcores=NS)

_sc_params = pltpu.CompilerParams()
if "needs_layout_passes" in pltpu.CompilerParams.__dataclass_fields__:
    _sc_params = dataclasses.replace(_sc_params, needs_layout_passes=False)
if "use_tc_tiling_on_sc" in pltpu.CompilerParams.__dataclass_fields__:
    _sc_params = dataclasses.replace(_sc_params, use_tc_tiling_on_sc=False)


@functools.partial(
    pl.kernel,
    out_type=(
        jax.ShapeDtypeStruct((NW, N_PAD), jnp.float32),
        jax.ShapeDtypeStruct((NW, N_PAD), jnp.float32),
    ),
    mesh=_mesh,
    scratch_types=[
        pltpu.VMEM((CHUNKS, CHUNK), jnp.int32),
        pltpu.VMEM((CHUNKS, CHUNK), jnp.int32),
        pltpu.VMEM((N_PAD,), jnp.float32),
        pltpu.VMEM((N_PAD,), jnp.float32),
    ],
    compiler_params=_sc_params,
)
def _sc_degrees(src_hbm, dst_hbm, od_hbm, id_hbm, sidx_v, didx_v, od_v, id_v):
    c = lax.axis_index("c")
    s = lax.axis_index("s")
    wid = c * NS + s
    pltpu.sync_copy(src_hbm.at[wid], sidx_v)
    pltpu.sync_copy(dst_hbm.at[wid], didx_v)
    zero16 = jnp.zeros((16,), jnp.float32)

    @pl.loop(0, N_PAD // 16)
    def _(i):
        od_v[pl.ds(i * 16, 16)] = zero16
        id_v[pl.ds(i * 16, 16)] = zero16

    ones = jnp.ones((16,), jnp.float32)

    @pl.loop(0, CHUNKS)
    def _(r):
        @pl.loop(0, CHUNK // 16)
        def _(k):
            plsc.addupdate_scatter(od_v, [sidx_v[r, pl.ds(k * 16, 16)]], ones)
            plsc.addupdate_scatter(id_v, [didx_v[r, pl.ds(k * 16, 16)]], ones)

    pltpu.sync_copy(od_v, od_hbm.at[wid])
    pltpu.sync_copy(id_v, id_hbm.at[wid])


@functools.partial(
    pl.kernel,
    out_type=jax.ShapeDtypeStruct((NC, 2, N_PAD, HALF), jnp.float32),
    mesh=_mesh,
    scratch_types=[
        pltpu.VMEM((CHUNKS, CHUNK), jnp.int32),
        pltpu.VMEM((CHUNKS, CHUNK), jnp.int32),
        pltpu.VMEM((CHUNK, HALF), jnp.float32),
        pltpu.VMEM((CHUNK, HALF), jnp.float32),
        pltpu.VMEM((CHUNK, HALF), jnp.float32),
        pltpu.VMEM_SHARED((N_PAD, HALF), jnp.float32),
        pltpu.SemaphoreType.DMA,
        pltpu.SemaphoreType.DMA,
        pltpu.SemaphoreType.DMA,
        pltpu.SemaphoreType.DMA,
    ],
    compiler_params=_sc_params,
)
def _sc_aggregate(feat0_hbm, feat1_hbm, src_hbm, dst_hbm, out_hbm,
                  sidx_v, didx_v, rows_a, rows_b, zrows, agg_sh,
                  sem_ga, sem_gb, sem_sa, sem_sb):
    c = lax.axis_index("c")
    s = lax.axis_index("s")
    wid = c * NS + s

    pltpu.sync_copy(src_hbm.at[wid], sidx_v)
    pltpu.sync_copy(dst_hbm.at[wid], didx_v)

    zero16 = jnp.zeros((16,), jnp.float32)

    @pl.loop(0, CHUNK)
    def _(r):
        @pl.loop(0, HALF // 16)
        def _(k):
            zrows[r, pl.ds(k * 16, 16)] = zero16

    for half, feat_hbm in ((0, feat0_hbm), (1, feat1_hbm)):
        @pl.loop(0, ROWS_PER_TILE // CHUNK)
        def _(j):
            pltpu.sync_copy(
                zrows, agg_sh.at[pl.ds(s * ROWS_PER_TILE + j * CHUNK, CHUNK)])

        plsc.subcore_barrier()

        @pl.loop(0, CHUNKS, step=2)
        def _(ch):
            ga = pltpu.async_copy(feat_hbm.at[sidx_v.at[ch]], rows_a, sem_ga)
            gb = pltpu.async_copy(feat_hbm.at[sidx_v.at[ch + 1]], rows_b, sem_gb)
            ga.wait()
            sa = pltpu.async_copy(rows_a, agg_sh.at[didx_v.at[ch]], sem_sa,
                                  add=True)
            gb.wait()
            sb = pltpu.async_copy(rows_b, agg_sh.at[didx_v.at[ch + 1]], sem_sb,
                                  add=True)
            sa.wait()
            sb.wait()

        plsc.subcore_barrier()
        pltpu.sync_copy(
            agg_sh.at[pl.ds(s * ROWS_PER_TILE, ROWS_PER_TILE)],
            out_hbm.at[c, half, pl.ds(s * ROWS_PER_TILE, ROWS_PER_TILE)])


def _tc_feat_body(h_ref, od_ref, o0_ref, o1_ref):
    od = jnp.sum(od_ref[...], axis=1)
    norm = lax.rsqrt(jnp.maximum(od, 1.0))
    f = h_ref[...] * norm[:, None]
    o0_ref[...] = f[:, :HALF]
    o1_ref[...] = f[:, HALF:]


def _tc_feat(h_pad, od_t):
    blk = 1024
    return pl.pallas_call(
        _tc_feat_body,
        grid=(N_PAD // blk,),
        in_specs=[
            pl.BlockSpec((blk, DIM), lambda i: (i, 0)),
            pl.BlockSpec((blk, NW), lambda i: (i, 0)),
        ],
        out_specs=[
            pl.BlockSpec((blk, HALF), lambda i: (i, 0)),
            pl.BlockSpec((blk, HALF), lambda i: (i, 0)),
        ],
        out_shape=[
            jax.ShapeDtypeStruct((N_PAD, HALF), jnp.float32),
            jax.ShapeDtypeStruct((N_PAD, HALF), jnp.float32),
        ],
    )(h_pad, od_t)


def _tc_out_body(agg_ref, id_ref, w_ref, b_ref, o_ref):
    ideg = jnp.sum(id_ref[...], axis=1)
    norm = lax.rsqrt(jnp.maximum(ideg, 1.0))
    a0 = (agg_ref[0, 0] + agg_ref[1, 0]) * norm[:, None]
    a1 = (agg_ref[0, 1] + agg_ref[1, 1]) * norm[:, None]
    w = w_ref[...]
    x = (jnp.dot(a0, w[:HALF, :], preferred_element_type=jnp.float32)
         + jnp.dot(a1, w[HALF:, :], preferred_element_type=jnp.float32)
         + b_ref[...])
    m = jnp.max(x, axis=1, keepdims=True)
    sh = x - m
    lse = jnp.log(jnp.sum(jnp.exp(sh), axis=1, keepdims=True))
    o_ref[...] = sh - lse


def _tc_out(agg_p, id_t, W, b2):
    blk = 1000
    return pl.pallas_call(
        _tc_out_body,
        grid=(N_NODES // blk,),
        in_specs=[
            pl.BlockSpec((NC, 2, blk, HALF), lambda i: (0, 0, i, 0)),
            pl.BlockSpec((blk, NW), lambda i: (i, 0)),
            pl.BlockSpec((DIM, DIM), lambda i: (0, 0)),
            pl.BlockSpec((1, DIM), lambda i: (0, 0)),
        ],
        out_specs=pl.BlockSpec((blk, DIM), lambda i: (i, 0)),
        out_shape=jax.ShapeDtypeStruct((N_NODES, DIM), jnp.float32),
    )(agg_p, id_t, W, b2)


def kernel(h, edge_index, W, b):
    src = edge_index[0].astype(jnp.int32).reshape(NW, EPW)
    dst = edge_index[1].astype(jnp.int32).reshape(NW, EPW)
    pad = N_NODES + (jnp.arange(PAD_E, dtype=jnp.int32) % 16)
    padw = jnp.broadcast_to(pad, (NW, PAD_E))
    src_p = jnp.concatenate([src, padw], axis=1).reshape(NW, CHUNKS, CHUNK)
    dst_p = jnp.concatenate([dst, padw], axis=1).reshape(NW, CHUNKS, CHUNK)

    od_p, id_p = _sc_degrees(src_p, dst_p)
    od_t = od_p.T
    id_t = id_p.T

    h_pad = jnp.pad(h, ((0, N_PAD - N_NODES), (0, 0)))
    feat0, feat1 = _tc_feat(h_pad, od_t)

    agg_p = _sc_aggregate(feat0, feat1, src_p, dst_p)

    return _tc_out(agg_p, id_t, W, b.reshape(1, DIM))


# speedup vs baseline: 9.8950x; 1.2075x over previous
import dataclasses
import functools

import jax
import jax.numpy as jnp
from jax import lax
from jax.experimental import pallas as pl
from jax.experimental.pallas import tpu as pltpu
from jax.experimental.pallas import tpu_sc as plsc

N_NODES = 10000
N_EDGES = 320000
DIM = 128
HALF = DIM // 2
NC = 2
NS = 16
NW = NC * NS
N_PAD = 10240
ROWS_PER_TILE = N_PAD // NS
CHUNK = 128
EPW = N_EDGES // NW
CHUNKS = 80
EPW_PAD = CHUNKS * CHUNK
PAD_E = EPW_PAD - EPW

_mesh = plsc.VectorSubcoreMesh(
    core_axis_name="c", subcore_axis_name="s", num_cores=NC, num_subcores=NS)

_sc_params = pltpu.CompilerParams()
if "needs_layout_passes" in pltpu.CompilerParams.__dataclass_fields__:
    _sc_params = dataclasses.replace(_sc_params, needs_layout_passes=False)
if "use_tc_tiling_on_sc" in pltpu.CompilerParams.__dataclass_fields__:
    _sc_params = dataclasses.replace(_sc_params, use_tc_tiling_on_sc=False)


@functools.partial(
    pl.kernel,
    out_type=(
        jax.ShapeDtypeStruct((NW, N_PAD), jnp.float32),
        jax.ShapeDtypeStruct((NW, N_PAD), jnp.float32),
    ),
    mesh=_mesh,
    scratch_types=[
        pltpu.VMEM((CHUNKS, CHUNK), jnp.int32),
        pltpu.VMEM((CHUNKS, CHUNK), jnp.int32),
        pltpu.VMEM((N_PAD,), jnp.float32),
        pltpu.VMEM((N_PAD,), jnp.float32),
    ],
    compiler_params=_sc_params,
)
def _sc_degrees(src_hbm, dst_hbm, od_hbm, id_hbm, sidx_v, didx_v, od_v, id_v):
    c = lax.axis_index("c")
    s = lax.axis_index("s")
    wid = c * NS + s
    pltpu.sync_copy(src_hbm.at[wid], sidx_v)
    pltpu.sync_copy(dst_hbm.at[wid], didx_v)
    zero16 = jnp.zeros((16,), jnp.float32)

    @pl.loop(0, N_PAD // 16)
    def _(i):
        od_v[pl.ds(i * 16, 16)] = zero16
        id_v[pl.ds(i * 16, 16)] = zero16

    ones = jnp.ones((16,), jnp.float32)

    @pl.loop(0, CHUNKS)
    def _(r):
        @pl.loop(0, CHUNK // 16)
        def _(k):
            plsc.addupdate_scatter(od_v, [sidx_v[r, pl.ds(k * 16, 16)]], ones)
            plsc.addupdate_scatter(id_v, [didx_v[r, pl.ds(k * 16, 16)]], ones)

    pltpu.sync_copy(od_v, od_hbm.at[wid])
    pltpu.sync_copy(id_v, id_hbm.at[wid])


NBUF = 4


@functools.partial(
    pl.kernel,
    out_type=jax.ShapeDtypeStruct((NC, 2, N_PAD, HALF), jnp.float32),
    mesh=_mesh,
    scratch_types=(
        [pltpu.VMEM((CHUNKS, CHUNK), jnp.int32)] * 2
        + [pltpu.VMEM((CHUNK, HALF), jnp.float32)] * (NBUF + 1)
        + [pltpu.VMEM_SHARED((N_PAD, HALF), jnp.float32)]
        + [pltpu.SemaphoreType.DMA] * (2 * NBUF)
    ),
    compiler_params=_sc_params,
)
def _sc_aggregate(feat0_hbm, feat1_hbm, src_hbm, dst_hbm, out_hbm, *refs):
    sidx_v, didx_v = refs[0], refs[1]
    bufs = refs[2:2 + NBUF]
    zrows = refs[2 + NBUF]
    agg_sh = refs[3 + NBUF]
    gsem = refs[4 + NBUF:4 + 2 * NBUF]
    ssem = refs[4 + 2 * NBUF:4 + 3 * NBUF]

    c = lax.axis_index("c")
    s = lax.axis_index("s")
    wid = c * NS + s

    pltpu.sync_copy(src_hbm.at[wid], sidx_v)
    pltpu.sync_copy(dst_hbm.at[wid], didx_v)

    zero16 = jnp.zeros((16,), jnp.float32)

    @pl.loop(0, CHUNK)
    def _(r):
        @pl.loop(0, HALF // 16)
        def _(k):
            zrows[r, pl.ds(k * 16, 16)] = zero16

    for half, feat_hbm in ((0, feat0_hbm), (1, feat1_hbm)):
        @pl.loop(0, ROWS_PER_TILE // CHUNK)
        def _(j):
            pltpu.sync_copy(
                zrows, agg_sh.at[pl.ds(s * ROWS_PER_TILE + j * CHUNK, CHUNK)])

        plsc.subcore_barrier()

        for j in range(NBUF):
            pltpu.async_copy(feat_hbm.at[sidx_v.at[j]], bufs[j], gsem[j])

        @pl.loop(0, CHUNKS - NBUF, step=NBUF)
        def _(ch):
            for j in range(NBUF):
                pltpu.make_async_copy(
                    feat_hbm.at[sidx_v.at[ch + j]], bufs[j], gsem[j]).wait()
                pltpu.async_copy(bufs[j], agg_sh.at[didx_v.at[ch + j]],
                                 ssem[j], add=True)
            for j in range(NBUF):
                pltpu.make_async_copy(
                    bufs[j], agg_sh.at[didx_v.at[ch + j]], ssem[j]).wait()
                pltpu.async_copy(
                    feat_hbm.at[sidx_v.at[ch + NBUF + j]], bufs[j], gsem[j])

        for j in range(NBUF):
            ch = CHUNKS - NBUF + j
            pltpu.make_async_copy(
                feat_hbm.at[sidx_v.at[ch]], bufs[j], gsem[j]).wait()
            pltpu.async_copy(bufs[j], agg_sh.at[didx_v.at[ch]],
                             ssem[j], add=True)
        for j in range(NBUF):
            ch = CHUNKS - NBUF + j
            pltpu.make_async_copy(
                bufs[j], agg_sh.at[didx_v.at[ch]], ssem[j]).wait()

        plsc.subcore_barrier()
        pltpu.sync_copy(
            agg_sh.at[pl.ds(s * ROWS_PER_TILE, ROWS_PER_TILE)],
            out_hbm.at[c, half, pl.ds(s * ROWS_PER_TILE, ROWS_PER_TILE)])


def _tc_feat_body(h_ref, od_ref, o0_ref, o1_ref):
    od = jnp.sum(od_ref[...], axis=1)
    norm = lax.rsqrt(jnp.maximum(od, 1.0))
    f = h_ref[...] * norm[:, None]
    o0_ref[...] = f[:, :HALF]
    o1_ref[...] = f[:, HALF:]


def _tc_feat(h_pad, od_t):
    blk = 1024
    return pl.pallas_call(
        _tc_feat_body,
        grid=(N_PAD // blk,),
        in_specs=[
            pl.BlockSpec((blk, DIM), lambda i: (i, 0)),
            pl.BlockSpec((blk, NW), lambda i: (i, 0)),
        ],
        out_specs=[
            pl.BlockSpec((blk, HALF), lambda i: (i, 0)),
            pl.BlockSpec((blk, HALF), lambda i: (i, 0)),
        ],
        out_shape=[
            jax.ShapeDtypeStruct((N_PAD, HALF), jnp.float32),
            jax.ShapeDtypeStruct((N_PAD, HALF), jnp.float32),
        ],
    )(h_pad, od_t)


def _tc_out_body(agg_ref, id_ref, w_ref, b_ref, o_ref):
    ideg = jnp.sum(id_ref[...], axis=1)
    norm = lax.rsqrt(jnp.maximum(ideg, 1.0))
    a0 = (agg_ref[0, 0] + agg_ref[1, 0]) * norm[:, None]
    a1 = (agg_ref[0, 1] + agg_ref[1, 1]) * norm[:, None]
    w = w_ref[...]
    x = (jnp.dot(a0, w[:HALF, :], preferred_element_type=jnp.float32)
         + jnp.dot(a1, w[HALF:, :], preferred_element_type=jnp.float32)
         + b_ref[...])
    m = jnp.max(x, axis=1, keepdims=True)
    sh = x - m
    lse = jnp.log(jnp.sum(jnp.exp(sh), axis=1, keepdims=True))
    o_ref[...] = sh - lse


def _tc_out(agg_p, id_t, W, b2):
    blk = 1000
    return pl.pallas_call(
        _tc_out_body,
        grid=(N_NODES // blk,),
        in_specs=[
            pl.BlockSpec((NC, 2, blk, HALF), lambda i: (0, 0, i, 0)),
            pl.BlockSpec((blk, NW), lambda i: (i, 0)),
            pl.BlockSpec((DIM, DIM), lambda i: (0, 0)),
            pl.BlockSpec((1, DIM), lambda i: (0, 0)),
        ],
        out_specs=pl.BlockSpec((blk, DIM), lambda i: (i, 0)),
        out_shape=jax.ShapeDtypeStruct((N_NODES, DIM), jnp.float32),
    )(agg_p, id_t, W, b2)


def kernel(h, edge_index, W, b):
    src = edge_index[0].astype(jnp.int32).reshape(NW, EPW)
    dst = edge_index[1].astype(jnp.int32).reshape(NW, EPW)
    pad = N_NODES + (jnp.arange(PAD_E, dtype=jnp.int32) % 16)
    padw = jnp.broadcast_to(pad, (NW, PAD_E))
    src_p = jnp.concatenate([src, padw], axis=1).reshape(NW, CHUNKS, CHUNK)
    dst_p = jnp.concatenate([dst, padw], axis=1).reshape(NW, CHUNKS, CHUNK)

    od_p, id_p = _sc_degrees(src_p, dst_p)
    od_t = od_p.T
    id_t = id_p.T

    h_pad = jnp.pad(h, ((0, N_PAD - N_NODES), (0, 0)))
    feat0, feat1 = _tc_feat(h_pad, od_t)

    agg_p = _sc_aggregate(feat0, feat1, src_p, dst_p)

    return _tc_out(agg_p, id_t, W, b.reshape(1, DIM))


# speedup vs baseline: 10.2515x; 1.0360x over previous
import dataclasses
import functools

import jax
import jax.numpy as jnp
from jax import lax
from jax.experimental import pallas as pl
from jax.experimental.pallas import tpu as pltpu
from jax.experimental.pallas import tpu_sc as plsc

N_NODES = 10000
N_EDGES = 320000
DIM = 128
HALF = DIM // 2
NC = 2
NS = 16
NW = NC * NS
N_PAD = 10240
ROWS_PER_TILE = N_PAD // NS
CHUNK = 128
EPW = N_EDGES // NW
CHUNKS = 80
EPW_PAD = CHUNKS * CHUNK
PAD_E = EPW_PAD - EPW

_mesh = plsc.VectorSubcoreMesh(
    core_axis_name="c", subcore_axis_name="s", num_cores=NC, num_subcores=NS)

_sc_params = pltpu.CompilerParams()
if "needs_layout_passes" in pltpu.CompilerParams.__dataclass_fields__:
    _sc_params = dataclasses.replace(_sc_params, needs_layout_passes=False)
if "use_tc_tiling_on_sc" in pltpu.CompilerParams.__dataclass_fields__:
    _sc_params = dataclasses.replace(_sc_params, use_tc_tiling_on_sc=False)


@functools.partial(
    pl.kernel,
    out_type=(
        jax.ShapeDtypeStruct((NW, N_PAD), jnp.float32),
        jax.ShapeDtypeStruct((NW, N_PAD), jnp.float32),
    ),
    mesh=_mesh,
    scratch_types=[
        pltpu.VMEM((CHUNKS, CHUNK), jnp.int32),
        pltpu.VMEM((CHUNKS, CHUNK), jnp.int32),
        pltpu.VMEM((N_PAD,), jnp.float32),
        pltpu.VMEM((N_PAD,), jnp.float32),
    ],
    compiler_params=_sc_params,
)
def _sc_degrees(src_hbm, dst_hbm, od_hbm, id_hbm, sidx_v, didx_v, od_v, id_v):
    c = lax.axis_index("c")
    s = lax.axis_index("s")
    wid = c * NS + s
    pltpu.sync_copy(src_hbm.at[wid], sidx_v)
    pltpu.sync_copy(dst_hbm.at[wid], didx_v)
    zero16 = jnp.zeros((16,), jnp.float32)

    @pl.loop(0, N_PAD // 16)
    def _(i):
        od_v[pl.ds(i * 16, 16)] = zero16
        id_v[pl.ds(i * 16, 16)] = zero16

    ones = jnp.ones((16,), jnp.float32)

    @pl.loop(0, CHUNKS)
    def _(r):
        @pl.loop(0, CHUNK // 16)
        def _(k):
            plsc.addupdate_scatter(od_v, [sidx_v[r, pl.ds(k * 16, 16)]], ones)
            plsc.addupdate_scatter(id_v, [didx_v[r, pl.ds(k * 16, 16)]], ones)

    pltpu.sync_copy(od_v, od_hbm.at[wid])
    pltpu.sync_copy(id_v, id_hbm.at[wid])


NBUF = 5


@functools.partial(
    pl.kernel,
    out_type=jax.ShapeDtypeStruct((NC, 2, N_PAD, HALF), jnp.float32),
    mesh=_mesh,
    scratch_types=(
        [pltpu.VMEM((CHUNKS, CHUNK), jnp.int32)] * 2
        + [pltpu.VMEM((CHUNK, HALF), jnp.float32)] * (NBUF + 1)
        + [pltpu.VMEM_SHARED((N_PAD, HALF), jnp.float32)]
        + [pltpu.SemaphoreType.DMA] * (2 * NBUF)
    ),
    compiler_params=_sc_params,
)
def _sc_aggregate(feat0_hbm, feat1_hbm, src_hbm, dst_hbm, out_hbm, *refs):
    sidx_v, didx_v = refs[0], refs[1]
    bufs = refs[2:2 + NBUF]
    zrows = refs[2 + NBUF]
    agg_sh = refs[3 + NBUF]
    gsem = refs[4 + NBUF:4 + 2 * NBUF]
    ssem = refs[4 + 2 * NBUF:4 + 3 * NBUF]

    c = lax.axis_index("c")
    s = lax.axis_index("s")
    wid = c * NS + s

    pltpu.sync_copy(src_hbm.at[wid], sidx_v)
    pltpu.sync_copy(dst_hbm.at[wid], didx_v)

    zero16 = jnp.zeros((16,), jnp.float32)

    @pl.loop(0, CHUNK)
    def _(r):
        @pl.loop(0, HALF // 16)
        def _(k):
            zrows[r, pl.ds(k * 16, 16)] = zero16

    for half, feat_hbm in ((0, feat0_hbm), (1, feat1_hbm)):
        @pl.loop(0, ROWS_PER_TILE // CHUNK)
        def _(j):
            pltpu.sync_copy(
                zrows, agg_sh.at[pl.ds(s * ROWS_PER_TILE + j * CHUNK, CHUNK)])

        plsc.subcore_barrier()

        for j in range(NBUF):
            pltpu.async_copy(feat_hbm.at[sidx_v.at[j]], bufs[j], gsem[j])

        @pl.loop(0, CHUNKS - NBUF, step=NBUF)
        def _(ch):
            for j in range(NBUF):
                pltpu.make_async_copy(
                    feat_hbm.at[sidx_v.at[ch + j]], bufs[j], gsem[j]).wait()
                pltpu.async_copy(bufs[j], agg_sh.at[didx_v.at[ch + j]],
                                 ssem[j], add=True)
            for j in range(NBUF):
                pltpu.make_async_copy(
                    bufs[j], agg_sh.at[didx_v.at[ch + j]], ssem[j]).wait()
                pltpu.async_copy(
                    feat_hbm.at[sidx_v.at[ch + NBUF + j]], bufs[j], gsem[j])

        for j in range(NBUF):
            ch = CHUNKS - NBUF + j
            pltpu.make_async_copy(
                feat_hbm.at[sidx_v.at[ch]], bufs[j], gsem[j]).wait()
            pltpu.async_copy(bufs[j], agg_sh.at[didx_v.at[ch]],
                             ssem[j], add=True)
        for j in range(NBUF):
            ch = CHUNKS - NBUF + j
            pltpu.make_async_copy(
                bufs[j], agg_sh.at[didx_v.at[ch]], ssem[j]).wait()

        plsc.subcore_barrier()
        pltpu.sync_copy(
            agg_sh.at[pl.ds(s * ROWS_PER_TILE, ROWS_PER_TILE)],
            out_hbm.at[c, half, pl.ds(s * ROWS_PER_TILE, ROWS_PER_TILE)])


def _tc_feat_body(h_ref, od_ref, o0_ref, o1_ref):
    od = jnp.sum(od_ref[...], axis=0)
    norm = lax.rsqrt(jnp.maximum(od, 1.0))
    f = h_ref[...] * norm[:, None]
    o0_ref[...] = f[:, :HALF]
    o1_ref[...] = f[:, HALF:]


def _tc_feat(h, od_p):
    blk = 1024
    return pl.pallas_call(
        _tc_feat_body,
        grid=(N_PAD // blk,),
        in_specs=[
            pl.BlockSpec((blk, DIM), lambda i: (i, 0)),
            pl.BlockSpec((NW, blk), lambda i: (0, i)),
        ],
        out_specs=[
            pl.BlockSpec((blk, HALF), lambda i: (i, 0)),
            pl.BlockSpec((blk, HALF), lambda i: (i, 0)),
        ],
        out_shape=[
            jax.ShapeDtypeStruct((N_PAD, HALF), jnp.float32),
            jax.ShapeDtypeStruct((N_PAD, HALF), jnp.float32),
        ],
    )(h, od_p)


def _tc_out_body(agg_ref, id_ref, w_ref, b_ref, o_ref):
    ideg = jnp.sum(id_ref[...], axis=0)
    norm = lax.rsqrt(jnp.maximum(ideg, 1.0))
    a0 = (agg_ref[0, 0] + agg_ref[1, 0]) * norm[:, None]
    a1 = (agg_ref[0, 1] + agg_ref[1, 1]) * norm[:, None]
    w = w_ref[...]
    x = (jnp.dot(a0, w[:HALF, :], preferred_element_type=jnp.float32)
         + jnp.dot(a1, w[HALF:, :], preferred_element_type=jnp.float32)
         + b_ref[...])
    m = jnp.max(x, axis=1, keepdims=True)
    sh = x - m
    lse = jnp.log(jnp.sum(jnp.exp(sh), axis=1, keepdims=True))
    o_ref[...] = sh - lse


def _tc_out(agg_p, id_p, W, b2):
    blk = 1024
    return pl.pallas_call(
        _tc_out_body,
        grid=(pl.cdiv(N_NODES, blk),),
        in_specs=[
            pl.BlockSpec((NC, 2, blk, HALF), lambda i: (0, 0, i, 0)),
            pl.BlockSpec((NW, blk), lambda i: (0, i)),
            pl.BlockSpec((DIM, DIM), lambda i: (0, 0)),
            pl.BlockSpec((1, DIM), lambda i: (0, 0)),
        ],
        out_specs=pl.BlockSpec((blk, DIM), lambda i: (i, 0)),
        out_shape=jax.ShapeDtypeStruct((N_NODES, DIM), jnp.float32),
    )(agg_p, id_p, W, b2)


def kernel(h, edge_index, W, b):
    src = edge_index[0].astype(jnp.int32).reshape(NW, EPW)
    dst = edge_index[1].astype(jnp.int32).reshape(NW, EPW)
    pad = N_NODES + (jnp.arange(PAD_E, dtype=jnp.int32) % 16)
    padw = jnp.broadcast_to(pad, (NW, PAD_E))
    src_p = jnp.concatenate([src, padw], axis=1).reshape(NW, CHUNKS, CHUNK)
    dst_p = jnp.concatenate([dst, padw], axis=1).reshape(NW, CHUNKS, CHUNK)

    od_p, id_p = _sc_degrees(src_p, dst_p)

    feat0, feat1 = _tc_feat(h, od_p)

    agg_p = _sc_aggregate(feat0, feat1, src_p, dst_p)

    return _tc_out(agg_p, id_p, W, b.reshape(1, DIM))


# speedup vs baseline: 10.5602x; 1.0301x over previous
import dataclasses
import functools

import jax
import jax.numpy as jnp
from jax import lax
from jax.experimental import pallas as pl
from jax.experimental.pallas import tpu as pltpu
from jax.experimental.pallas import tpu_sc as plsc

N_NODES = 10000
N_EDGES = 320000
DIM = 128
HALF = DIM // 2
NC = 2
NS = 16
NW = NC * NS
N_PAD = 10240
ROWS_PER_TILE = N_PAD // NS
CHUNK = 128
EPW = N_EDGES // NW
CHUNKS = 80
EPW_PAD = CHUNKS * CHUNK
PAD_E = EPW_PAD - EPW

_mesh = plsc.VectorSubcoreMesh(
    core_axis_name="c", subcore_axis_name="s", num_cores=NC, num_subcores=NS)

_sc_params = pltpu.CompilerParams()
if "needs_layout_passes" in pltpu.CompilerParams.__dataclass_fields__:
    _sc_params = dataclasses.replace(_sc_params, needs_layout_passes=False)
if "use_tc_tiling_on_sc" in pltpu.CompilerParams.__dataclass_fields__:
    _sc_params = dataclasses.replace(_sc_params, use_tc_tiling_on_sc=False)


@functools.partial(
    pl.kernel,
    out_type=(
        jax.ShapeDtypeStruct((NW, N_PAD), jnp.float32),
        jax.ShapeDtypeStruct((NW, N_PAD), jnp.float32),
    ),
    mesh=_mesh,
    scratch_types=[
        pltpu.VMEM((CHUNKS, CHUNK), jnp.int32),
        pltpu.VMEM((CHUNKS, CHUNK), jnp.int32),
        pltpu.VMEM((N_PAD,), jnp.float32),
        pltpu.VMEM((N_PAD,), jnp.float32),
    ],
    compiler_params=_sc_params,
)
def _sc_degrees(src_hbm, dst_hbm, od_hbm, id_hbm, sidx_v, didx_v, od_v, id_v):
    c = lax.axis_index("c")
    s = lax.axis_index("s")
    wid = c * NS + s
    pltpu.sync_copy(src_hbm.at[wid], sidx_v)
    pltpu.sync_copy(dst_hbm.at[wid], didx_v)
    zero16 = jnp.zeros((16,), jnp.float32)

    @pl.loop(0, N_PAD // 16)
    def _(i):
        od_v[pl.ds(i * 16, 16)] = zero16
        id_v[pl.ds(i * 16, 16)] = zero16

    ones = jnp.ones((16,), jnp.float32)

    @pl.loop(0, CHUNKS)
    def _(r):
        for k in range(CHUNK // 16):
            plsc.addupdate_scatter(od_v, [sidx_v[r, pl.ds(k * 16, 16)]], ones)
            plsc.addupdate_scatter(id_v, [didx_v[r, pl.ds(k * 16, 16)]], ones)

    pltpu.sync_copy(od_v, od_hbm.at[wid])
    pltpu.sync_copy(id_v, id_hbm.at[wid])


NBUF = 5


@functools.partial(
    pl.kernel,
    out_type=jax.ShapeDtypeStruct((NC, 2, N_PAD, HALF), jnp.float32),
    mesh=_mesh,
    scratch_types=(
        [pltpu.VMEM((CHUNKS, CHUNK), jnp.int32)] * 2
        + [pltpu.VMEM((CHUNK, HALF), jnp.float32)] * (NBUF + 1)
        + [pltpu.VMEM_SHARED((N_PAD, HALF), jnp.float32)]
        + [pltpu.SemaphoreType.DMA] * (2 * NBUF)
    ),
    compiler_params=_sc_params,
)
def _sc_aggregate(feat0_hbm, feat1_hbm, src_hbm, dst_hbm, out_hbm, *refs):
    sidx_v, didx_v = refs[0], refs[1]
    bufs = refs[2:2 + NBUF]
    zrows = refs[2 + NBUF]
    agg_sh = refs[3 + NBUF]
    gsem = refs[4 + NBUF:4 + 2 * NBUF]
    ssem = refs[4 + 2 * NBUF:4 + 3 * NBUF]

    c = lax.axis_index("c")
    s = lax.axis_index("s")
    wid = c * NS + s

    pltpu.sync_copy(src_hbm.at[wid], sidx_v)
    pltpu.sync_copy(dst_hbm.at[wid], didx_v)

    zero16 = jnp.zeros((16,), jnp.float32)

    @pl.loop(0, CHUNK)
    def _(r):
        @pl.loop(0, HALF // 16)
        def _(k):
            zrows[r, pl.ds(k * 16, 16)] = zero16

    for half, feat_hbm in ((0, feat0_hbm), (1, feat1_hbm)):
        @pl.loop(0, ROWS_PER_TILE // CHUNK)
        def _(j):
            pltpu.sync_copy(
                zrows, agg_sh.at[pl.ds(s * ROWS_PER_TILE + j * CHUNK, CHUNK)])

        plsc.subcore_barrier()

        for j in range(NBUF):
            pltpu.async_copy(feat_hbm.at[sidx_v.at[j]], bufs[j], gsem[j])

        @pl.loop(0, CHUNKS - NBUF, step=NBUF)
        def _(ch):
            for j in range(NBUF):
                pltpu.make_async_copy(
                    feat_hbm.at[sidx_v.at[ch + j]], bufs[j], gsem[j]).wait()
                pltpu.async_copy(bufs[j], agg_sh.at[didx_v.at[ch + j]],
                                 ssem[j], add=True)
            for j in range(NBUF):
                pltpu.make_async_copy(
                    bufs[j], agg_sh.at[didx_v.at[ch + j]], ssem[j]).wait()
                pltpu.async_copy(
                    feat_hbm.at[sidx_v.at[ch + NBUF + j]], bufs[j], gsem[j])

        for j in range(NBUF):
            ch = CHUNKS - NBUF + j
            pltpu.make_async_copy(
                feat_hbm.at[sidx_v.at[ch]], bufs[j], gsem[j]).wait()
            pltpu.async_copy(bufs[j], agg_sh.at[didx_v.at[ch]],
                             ssem[j], add=True)
        for j in range(NBUF):
            ch = CHUNKS - NBUF + j
            pltpu.make_async_copy(
                bufs[j], agg_sh.at[didx_v.at[ch]], ssem[j]).wait()

        plsc.subcore_barrier()
        pltpu.sync_copy(
            agg_sh.at[pl.ds(s * ROWS_PER_TILE, ROWS_PER_TILE)],
            out_hbm.at[c, half, pl.ds(s * ROWS_PER_TILE, ROWS_PER_TILE)])


def _tc_feat_body(h_ref, od_ref, o0_ref, o1_ref):
    od = jnp.sum(od_ref[...], axis=0)
    norm = lax.rsqrt(jnp.maximum(od, 1.0))
    f = h_ref[...] * norm[:, None]
    o0_ref[...] = f[:, :HALF]
    o1_ref[...] = f[:, HALF:]


def _tc_feat(h, od_p):
    blk = 1024
    return pl.pallas_call(
        _tc_feat_body,
        grid=(N_PAD // blk,),
        in_specs=[
            pl.BlockSpec((blk, DIM), lambda i: (i, 0)),
            pl.BlockSpec((NW, blk), lambda i: (0, i)),
        ],
        out_specs=[
            pl.BlockSpec((blk, HALF), lambda i: (i, 0)),
            pl.BlockSpec((blk, HALF), lambda i: (i, 0)),
        ],
        out_shape=[
            jax.ShapeDtypeStruct((N_PAD, HALF), jnp.float32),
            jax.ShapeDtypeStruct((N_PAD, HALF), jnp.float32),
        ],
    )(h, od_p)


def _tc_out_body(agg_ref, id_ref, w_ref, b_ref, o_ref):
    ideg = jnp.sum(id_ref[...], axis=0)
    norm = lax.rsqrt(jnp.maximum(ideg, 1.0))
    a0 = (agg_ref[0, 0] + agg_ref[1, 0]) * norm[:, None]
    a1 = (agg_ref[0, 1] + agg_ref[1, 1]) * norm[:, None]
    w = w_ref[...]
    x = (jnp.dot(a0, w[:HALF, :], preferred_element_type=jnp.float32)
         + jnp.dot(a1, w[HALF:, :], preferred_element_type=jnp.float32)
         + b_ref[...])
    m = jnp.max(x, axis=1, keepdims=True)
    sh = x - m
    lse = jnp.log(jnp.sum(jnp.exp(sh), axis=1, keepdims=True))
    o_ref[...] = sh - lse


def _tc_out(agg_p, id_p, W, b2):
    blk = 1024
    return pl.pallas_call(
        _tc_out_body,
        grid=(pl.cdiv(N_NODES, blk),),
        in_specs=[
            pl.BlockSpec((NC, 2, blk, HALF), lambda i: (0, 0, i, 0)),
            pl.BlockSpec((NW, blk), lambda i: (0, i)),
            pl.BlockSpec((DIM, DIM), lambda i: (0, 0)),
            pl.BlockSpec((1, DIM), lambda i: (0, 0)),
        ],
        out_specs=pl.BlockSpec((blk, DIM), lambda i: (i, 0)),
        out_shape=jax.ShapeDtypeStruct((N_NODES, DIM), jnp.float32),
    )(agg_p, id_p, W, b2)


def kernel(h, edge_index, W, b):
    ei = edge_index.astype(jnp.int32)
    pad = N_NODES + (jnp.arange(NW * PAD_E, dtype=jnp.int32) % 16)
    src_p = jnp.concatenate([ei[0], pad]).reshape(NW, CHUNKS, CHUNK)
    dst_p = jnp.concatenate([ei[1], pad]).reshape(NW, CHUNKS, CHUNK)

    od_p, id_p = _sc_degrees(src_p, dst_p)

    feat0, feat1 = _tc_feat(h, od_p)

    agg_p = _sc_aggregate(feat0, feat1, src_p, dst_p)

    return _tc_out(agg_p, id_p, W, b.reshape(1, DIM))


# speedup vs baseline: 12.0968x; 1.1455x over previous
import dataclasses
import functools

import jax
import jax.numpy as jnp
from jax import lax
from jax.experimental import pallas as pl
from jax.experimental.pallas import tpu as pltpu
from jax.experimental.pallas import tpu_sc as plsc

N_NODES = 10000
N_EDGES = 320000
DIM = 128
HALF = DIM // 2
NC = 2
NS = 16
NW = NC * NS
N_PAD = 10240
ROWS_PER_TILE = N_PAD // NS
CHUNK = 128
EPW = N_EDGES // NW
CHUNKS = 80
EPW_PAD = CHUNKS * CHUNK
PAD_E = EPW_PAD - EPW

_mesh = plsc.VectorSubcoreMesh(
    core_axis_name="c", subcore_axis_name="s", num_cores=NC, num_subcores=NS)

_sc_params = pltpu.CompilerParams()
if "needs_layout_passes" in pltpu.CompilerParams.__dataclass_fields__:
    _sc_params = dataclasses.replace(_sc_params, needs_layout_passes=False)
if "use_tc_tiling_on_sc" in pltpu.CompilerParams.__dataclass_fields__:
    _sc_params = dataclasses.replace(_sc_params, use_tc_tiling_on_sc=False)


@functools.partial(
    pl.kernel,
    out_type=(
        jax.ShapeDtypeStruct((NW, N_PAD), jnp.float32),
        jax.ShapeDtypeStruct((NW, N_PAD), jnp.float32),
    ),
    mesh=_mesh,
    scratch_types=[
        pltpu.VMEM((CHUNKS, CHUNK), jnp.int32),
        pltpu.VMEM((CHUNKS, CHUNK), jnp.int32),
        pltpu.VMEM((N_PAD,), jnp.float32),
        pltpu.VMEM((N_PAD,), jnp.float32),
    ],
    compiler_params=_sc_params,
)
def _sc_degrees(src_hbm, dst_hbm, od_hbm, id_hbm, sidx_v, didx_v, od_v, id_v):
    c = lax.axis_index("c")
    s = lax.axis_index("s")
    wid = c * NS + s
    pltpu.sync_copy(src_hbm.at[pl.ds(wid * CHUNKS, CHUNKS)], sidx_v)
    pltpu.sync_copy(dst_hbm.at[pl.ds(wid * CHUNKS, CHUNKS)], didx_v)
    zero16 = jnp.zeros((16,), jnp.float32)

    @pl.loop(0, N_PAD // 16)
    def _(i):
        od_v[pl.ds(i * 16, 16)] = zero16
        id_v[pl.ds(i * 16, 16)] = zero16

    ones = jnp.ones((16,), jnp.float32)

    @pl.loop(0, CHUNKS)
    def _(r):
        for k in range(CHUNK // 16):
            plsc.addupdate_scatter(od_v, [sidx_v[r, pl.ds(k * 16, 16)]], ones)
            plsc.addupdate_scatter(id_v, [didx_v[r, pl.ds(k * 16, 16)]], ones)

    pltpu.sync_copy(od_v, od_hbm.at[wid])
    pltpu.sync_copy(id_v, id_hbm.at[wid])


NBUF = 5
CHUNKS2 = NW * CHUNKS // NS


@functools.partial(
    pl.kernel,
    out_type=jax.ShapeDtypeStruct((NC, N_PAD, HALF), jnp.float32),
    mesh=_mesh,
    scratch_types=(
        [pltpu.VMEM((CHUNKS2, CHUNK), jnp.int32)] * 2
        + [pltpu.VMEM((CHUNK, HALF), jnp.float32)] * (NBUF + 1)
        + [pltpu.VMEM_SHARED((N_PAD, HALF), jnp.float32)]
        + [pltpu.SemaphoreType.DMA] * (2 * NBUF)
    ),
    compiler_params=_sc_params,
)
def _sc_aggregate(feat_hbm, src_hbm, dst_hbm, out_hbm, *refs):
    sidx_v, didx_v = refs[0], refs[1]
    bufs = refs[2:2 + NBUF]
    zrows = refs[2 + NBUF]
    agg_sh = refs[3 + NBUF]
    gsem = refs[4 + NBUF:4 + 2 * NBUF]
    ssem = refs[4 + 2 * NBUF:4 + 3 * NBUF]

    c = lax.axis_index("c")
    s = lax.axis_index("s")

    pltpu.sync_copy(src_hbm.at[pl.ds(s * CHUNKS2, CHUNKS2)], sidx_v)
    pltpu.sync_copy(dst_hbm.at[pl.ds(s * CHUNKS2, CHUNKS2)], didx_v)

    zero16 = jnp.zeros((16,), jnp.float32)

    @pl.loop(0, CHUNK)
    def _(r):
        for k in range(HALF // 16):
            zrows[r, pl.ds(k * 16, 16)] = zero16

    @pl.loop(0, ROWS_PER_TILE // CHUNK)
    def _(j):
        pltpu.sync_copy(
            zrows, agg_sh.at[pl.ds(s * ROWS_PER_TILE + j * CHUNK, CHUNK)])

    plsc.subcore_barrier()

    fsrc = feat_hbm.at[c]

    for j in range(NBUF):
        pltpu.async_copy(fsrc.at[sidx_v.at[j]], bufs[j], gsem[j])

    @pl.loop(0, CHUNKS2 - NBUF, step=NBUF)
    def _(ch):
        for j in range(NBUF):
            pltpu.make_async_copy(
                fsrc.at[sidx_v.at[ch + j]], bufs[j], gsem[j]).wait()
            pltpu.async_copy(bufs[j], agg_sh.at[didx_v.at[ch + j]],
                             ssem[j], add=True)
        for j in range(NBUF):
            pltpu.make_async_copy(
                bufs[j], agg_sh.at[didx_v.at[ch + j]], ssem[j]).wait()
            pltpu.async_copy(
                fsrc.at[sidx_v.at[ch + NBUF + j]], bufs[j], gsem[j])

    for j in range(NBUF):
        ch = CHUNKS2 - NBUF + j
        pltpu.make_async_copy(
            fsrc.at[sidx_v.at[ch]], bufs[j], gsem[j]).wait()
        pltpu.async_copy(bufs[j], agg_sh.at[didx_v.at[ch]],
                         ssem[j], add=True)
    for j in range(NBUF):
        ch = CHUNKS2 - NBUF + j
        pltpu.make_async_copy(
            bufs[j], agg_sh.at[didx_v.at[ch]], ssem[j]).wait()

    plsc.subcore_barrier()
    pltpu.sync_copy(
        agg_sh.at[pl.ds(s * ROWS_PER_TILE, ROWS_PER_TILE)],
        out_hbm.at[c, pl.ds(s * ROWS_PER_TILE, ROWS_PER_TILE)])


def _tc_feat_body(h_ref, od_ref, o_ref):
    od = jnp.sum(od_ref[...], axis=0)
    norm = lax.rsqrt(jnp.maximum(od, 1.0))
    f = h_ref[...] * norm[:, None]
    o_ref[0] = f[:, :HALF]
    o_ref[1] = f[:, HALF:]


def _tc_feat(h, od_p):
    blk = 1024
    return pl.pallas_call(
        _tc_feat_body,
        grid=(N_PAD // blk,),
        in_specs=[
            pl.BlockSpec((blk, DIM), lambda i: (i, 0)),
            pl.BlockSpec((NW, blk), lambda i: (0, i)),
        ],
        out_specs=pl.BlockSpec((NC, blk, HALF), lambda i: (0, i, 0)),
        out_shape=jax.ShapeDtypeStruct((NC, N_PAD, HALF), jnp.float32),
    )(h, od_p)


def _tc_out_body(agg_ref, id_ref, w_ref, b_ref, o_ref):
    ideg = jnp.sum(id_ref[...], axis=0)
    norm = lax.rsqrt(jnp.maximum(ideg, 1.0))
    a0 = agg_ref[0] * norm[:, None]
    a1 = agg_ref[1] * norm[:, None]
    w = w_ref[...]
    x = (jnp.dot(a0, w[:HALF, :], preferred_element_type=jnp.float32)
         + jnp.dot(a1, w[HALF:, :], preferred_element_type=jnp.float32)
         + b_ref[...])
    m = jnp.max(x, axis=1, keepdims=True)
    sh = x - m
    lse = jnp.log(jnp.sum(jnp.exp(sh), axis=1, keepdims=True))
    o_ref[...] = sh - lse


def _tc_out(agg_p, id_p, W, b2):
    blk = 1024
    return pl.pallas_call(
        _tc_out_body,
        grid=(pl.cdiv(N_NODES, blk),),
        in_specs=[
            pl.BlockSpec((NC, blk, HALF), lambda i: (0, i, 0)),
            pl.BlockSpec((NW, blk), lambda i: (0, i)),
            pl.BlockSpec((DIM, DIM), lambda i: (0, 0)),
            pl.BlockSpec((1, DIM), lambda i: (0, 0)),
        ],
        out_specs=pl.BlockSpec((blk, DIM), lambda i: (i, 0)),
        out_shape=jax.ShapeDtypeStruct((N_NODES, DIM), jnp.float32),
    )(agg_p, id_p, W, b2)


def kernel(h, edge_index, W, b):
    ei = edge_index.astype(jnp.int32)
    pad = N_NODES + (jnp.arange(NW * PAD_E, dtype=jnp.int32) % 16)
    src_p = jnp.concatenate([ei[0], pad]).reshape(NW * CHUNKS, CHUNK)
    dst_p = jnp.concatenate([ei[1], pad]).reshape(NW * CHUNKS, CHUNK)

    od_p, id_p = _sc_degrees(src_p, dst_p)

    feat = _tc_feat(h, od_p)

    agg_p = _sc_aggregate(feat, src_p, dst_p)

    return _tc_out(agg_p, id_p, W, b.reshape(1, DIM))


# speedup vs baseline: 12.1489x; 1.0043x over previous
import dataclasses
import functools

import jax
import jax.numpy as jnp
from jax import lax
from jax.experimental import pallas as pl
from jax.experimental.pallas import tpu as pltpu
from jax.experimental.pallas import tpu_sc as plsc

N_NODES = 10000
N_EDGES = 320000
DIM = 128
HALF = DIM // 2
NC = 2
NS = 16
NW = NC * NS
N_PAD = 10240
ROWS_PER_TILE = N_PAD // NS
CHUNK = 128
EPW = N_EDGES // NW
CHUNKS = 80
EPW_PAD = CHUNKS * CHUNK
PAD_E = EPW_PAD - EPW

_mesh = plsc.VectorSubcoreMesh(
    core_axis_name="c", subcore_axis_name="s", num_cores=NC, num_subcores=NS)

_sc_params = pltpu.CompilerParams()
if "needs_layout_passes" in pltpu.CompilerParams.__dataclass_fields__:
    _sc_params = dataclasses.replace(_sc_params, needs_layout_passes=False)
if "use_tc_tiling_on_sc" in pltpu.CompilerParams.__dataclass_fields__:
    _sc_params = dataclasses.replace(_sc_params, use_tc_tiling_on_sc=False)


@functools.partial(
    pl.kernel,
    out_type=(
        jax.ShapeDtypeStruct((NW, N_PAD), jnp.float32),
        jax.ShapeDtypeStruct((NW, N_PAD), jnp.float32),
    ),
    mesh=_mesh,
    scratch_types=[
        pltpu.VMEM((CHUNKS, CHUNK), jnp.int32),
        pltpu.VMEM((CHUNKS, CHUNK), jnp.int32),
        pltpu.VMEM((N_PAD,), jnp.float32),
        pltpu.VMEM((N_PAD,), jnp.float32),
    ],
    compiler_params=_sc_params,
)
def _sc_degrees(idx_hbm, od_hbm, id_hbm, sidx_v, didx_v, od_v, id_v):
    c = lax.axis_index("c")
    s = lax.axis_index("s")
    wid = c * NS + s
    pltpu.sync_copy(idx_hbm.at[0, pl.ds(wid * CHUNKS, CHUNKS)], sidx_v)
    pltpu.sync_copy(idx_hbm.at[1, pl.ds(wid * CHUNKS, CHUNKS)], didx_v)
    zero16 = jnp.zeros((16,), jnp.float32)

    @pl.loop(0, N_PAD // 64)
    def _(i):
        for k in range(4):
            od_v[pl.ds(i * 64 + k * 16, 16)] = zero16
            id_v[pl.ds(i * 64 + k * 16, 16)] = zero16

    ones = jnp.ones((16,), jnp.float32)

    @pl.loop(0, CHUNKS)
    def _(r):
        for k in range(CHUNK // 16):
            plsc.addupdate_scatter(od_v, [sidx_v[r, pl.ds(k * 16, 16)]], ones)
            plsc.addupdate_scatter(id_v, [didx_v[r, pl.ds(k * 16, 16)]], ones)

    pltpu.sync_copy(od_v, od_hbm.at[wid])
    pltpu.sync_copy(id_v, id_hbm.at[wid])


NBUF = 5
CHUNKS2 = NW * CHUNKS // NS


@functools.partial(
    pl.kernel,
    out_type=jax.ShapeDtypeStruct((NC, N_PAD, HALF), jnp.float32),
    mesh=_mesh,
    scratch_types=(
        [pltpu.VMEM((CHUNKS2, CHUNK), jnp.int32)] * 2
        + [pltpu.VMEM((CHUNK, HALF), jnp.float32)] * (NBUF + 1)
        + [pltpu.VMEM_SHARED((N_PAD, HALF), jnp.float32)]
        + [pltpu.SemaphoreType.DMA] * (2 * NBUF)
    ),
    compiler_params=_sc_params,
)
def _sc_aggregate(feat_hbm, idx_hbm, out_hbm, *refs):
    sidx_v, didx_v = refs[0], refs[1]
    bufs = refs[2:2 + NBUF]
    zrows = refs[2 + NBUF]
    agg_sh = refs[3 + NBUF]
    gsem = refs[4 + NBUF:4 + 2 * NBUF]
    ssem = refs[4 + 2 * NBUF:4 + 3 * NBUF]

    c = lax.axis_index("c")
    s = lax.axis_index("s")

    pltpu.sync_copy(idx_hbm.at[0, pl.ds(s * CHUNKS2, CHUNKS2)], sidx_v)
    pltpu.sync_copy(idx_hbm.at[1, pl.ds(s * CHUNKS2, CHUNKS2)], didx_v)

    zero16 = jnp.zeros((16,), jnp.float32)

    @pl.loop(0, CHUNK)
    def _(r):
        for k in range(HALF // 16):
            zrows[r, pl.ds(k * 16, 16)] = zero16

    @pl.loop(0, ROWS_PER_TILE // CHUNK)
    def _(j):
        pltpu.sync_copy(
            zrows, agg_sh.at[pl.ds(s * ROWS_PER_TILE + j * CHUNK, CHUNK)])

    plsc.subcore_barrier()

    fsrc = feat_hbm.at[c]

    for j in range(NBUF):
        pltpu.async_copy(fsrc.at[sidx_v.at[j]], bufs[j], gsem[j])

    @pl.loop(0, CHUNKS2 - NBUF, step=NBUF)
    def _(ch):
        for j in range(NBUF):
            pltpu.make_async_copy(
                fsrc.at[sidx_v.at[ch + j]], bufs[j], gsem[j]).wait()
            pltpu.async_copy(bufs[j], agg_sh.at[didx_v.at[ch + j]],
                             ssem[j], add=True)
        for j in range(NBUF):
            pltpu.make_async_copy(
                bufs[j], agg_sh.at[didx_v.at[ch + j]], ssem[j]).wait()
            pltpu.async_copy(
                fsrc.at[sidx_v.at[ch + NBUF + j]], bufs[j], gsem[j])

    for j in range(NBUF):
        ch = CHUNKS2 - NBUF + j
        pltpu.make_async_copy(
            fsrc.at[sidx_v.at[ch]], bufs[j], gsem[j]).wait()
        pltpu.async_copy(bufs[j], agg_sh.at[didx_v.at[ch]],
                         ssem[j], add=True)
    for j in range(NBUF):
        ch = CHUNKS2 - NBUF + j
        pltpu.make_async_copy(
            bufs[j], agg_sh.at[didx_v.at[ch]], ssem[j]).wait()

    plsc.subcore_barrier()
    pltpu.sync_copy(
        agg_sh.at[pl.ds(s * ROWS_PER_TILE, ROWS_PER_TILE)],
        out_hbm.at[c, pl.ds(s * ROWS_PER_TILE, ROWS_PER_TILE)])


def _tc_feat_body(h_ref, od_ref, o_ref):
    od = jnp.sum(od_ref[...], axis=0)
    norm = lax.rsqrt(jnp.maximum(od, 1.0))
    o_ref[...] = h_ref[...] * norm[:, None]


def _tc_feat(h, od_p):
    blk = 1024
    return pl.pallas_call(
        _tc_feat_body,
        grid=(N_PAD // blk,),
        in_specs=[
            pl.BlockSpec((blk, DIM), lambda i: (i, 0)),
            pl.BlockSpec((NW, blk), lambda i: (0, i)),
        ],
        out_specs=pl.BlockSpec((blk, DIM), lambda i: (i, 0)),
        out_shape=jax.ShapeDtypeStruct((N_PAD, DIM), jnp.float32),
    )(h, od_p)


def _tc_out_body(agg_ref, id_ref, w_ref, b_ref, o_ref):
    ideg = jnp.sum(id_ref[...], axis=0)
    norm = lax.rsqrt(jnp.maximum(ideg, 1.0))
    a0 = agg_ref[0] * norm[:, None]
    a1 = agg_ref[1] * norm[:, None]
    w = w_ref[...]
    x = (jnp.dot(a0, w[:HALF, :], preferred_element_type=jnp.float32)
         + jnp.dot(a1, w[HALF:, :], preferred_element_type=jnp.float32)
         + b_ref[...])
    m = jnp.max(x, axis=1, keepdims=True)
    sh = x - m
    lse = jnp.log(jnp.sum(jnp.exp(sh), axis=1, keepdims=True))
    o_ref[...] = sh - lse


def _tc_out(agg_p, id_p, W, b2):
    blk = 1024
    return pl.pallas_call(
        _tc_out_body,
        grid=(pl.cdiv(N_NODES, blk),),
        in_specs=[
            pl.BlockSpec((NC, blk, HALF), lambda i: (0, i, 0)),
            pl.BlockSpec((NW, blk), lambda i: (0, i)),
            pl.BlockSpec((DIM, DIM), lambda i: (0, 0)),
            pl.BlockSpec((1, DIM), lambda i: (0, 0)),
        ],
        out_specs=pl.BlockSpec((blk, DIM), lambda i: (i, 0)),
        out_shape=jax.ShapeDtypeStruct((N_NODES, DIM), jnp.float32),
    )(agg_p, id_p, W, b2)


def kernel(h, edge_index, W, b):
    ei3 = edge_index.astype(jnp.int32).reshape(2, N_EDGES // CHUNK, CHUNK)
    pad = N_NODES + (jnp.arange(NW * PAD_E, dtype=jnp.int32) % 16)
    padc = jnp.broadcast_to(
        pad.reshape(1, NW * PAD_E // CHUNK, CHUNK),
        (2, NW * PAD_E // CHUNK, CHUNK))
    idx3 = jnp.concatenate([ei3, padc], axis=1)

    od_p, id_p = _sc_degrees(idx3)

    featw = _tc_feat(h, od_p)
    feat = featw.reshape(N_PAD, NC, HALF).transpose(1, 0, 2)

    agg_p = _sc_aggregate(feat, idx3)

    return _tc_out(agg_p, id_p, W, b.reshape(1, DIM))


# speedup vs baseline: 12.1960x; 1.0039x over previous
import dataclasses
import functools

import jax
import jax.numpy as jnp
from jax import lax
from jax.experimental import pallas as pl
from jax.experimental.pallas import tpu as pltpu
from jax.experimental.pallas import tpu_sc as plsc

N_NODES = 10000
N_EDGES = 320000
DIM = 128
HALF = DIM // 2
NC = 2
NS = 16
NW = NC * NS
N_PAD = 10240
ROWS_PER_TILE = N_PAD // NS
CHUNK = 128
EPW = N_EDGES // NW
CHUNKS = 80
EPW_PAD = CHUNKS * CHUNK
PAD_E = EPW_PAD - EPW

_mesh = plsc.VectorSubcoreMesh(
    core_axis_name="c", subcore_axis_name="s", num_cores=NC, num_subcores=NS)

_sc_params = pltpu.CompilerParams()
if "needs_layout_passes" in pltpu.CompilerParams.__dataclass_fields__:
    _sc_params = dataclasses.replace(_sc_params, needs_layout_passes=False)
if "use_tc_tiling_on_sc" in pltpu.CompilerParams.__dataclass_fields__:
    _sc_params = dataclasses.replace(_sc_params, use_tc_tiling_on_sc=False)


@functools.partial(
    pl.kernel,
    out_type=(
        jax.ShapeDtypeStruct((NW, N_PAD), jnp.float32),
        jax.ShapeDtypeStruct((NW, N_PAD), jnp.float32),
    ),
    mesh=_mesh,
    scratch_types=[
        pltpu.VMEM((CHUNKS, CHUNK), jnp.int32),
        pltpu.VMEM((CHUNKS, CHUNK), jnp.int32),
        pltpu.VMEM((N_PAD,), jnp.float32),
        pltpu.VMEM((N_PAD,), jnp.float32),
    ],
    compiler_params=_sc_params,
)
def _sc_degrees(idx_hbm, od_hbm, id_hbm, sidx_v, didx_v, od_v, id_v):
    c = lax.axis_index("c")
    s = lax.axis_index("s")
    wid = c * NS + s
    pltpu.sync_copy(idx_hbm.at[0, pl.ds(wid * CHUNKS, CHUNKS)], sidx_v)
    pltpu.sync_copy(idx_hbm.at[1, pl.ds(wid * CHUNKS, CHUNKS)], didx_v)
    zero16 = jnp.zeros((16,), jnp.float32)

    @pl.loop(0, N_PAD // 64)
    def _(i):
        for k in range(4):
            od_v[pl.ds(i * 64 + k * 16, 16)] = zero16
            id_v[pl.ds(i * 64 + k * 16, 16)] = zero16

    ones = jnp.ones((16,), jnp.float32)

    @pl.loop(0, CHUNKS)
    def _(r):
        for k in range(CHUNK // 16):
            plsc.addupdate_scatter(od_v, [sidx_v[r, pl.ds(k * 16, 16)]], ones)
            plsc.addupdate_scatter(id_v, [didx_v[r, pl.ds(k * 16, 16)]], ones)

    pltpu.sync_copy(od_v, od_hbm.at[wid])
    pltpu.sync_copy(id_v, id_hbm.at[wid])


NBUF = 5
CHUNKS2 = NW * CHUNKS // NS


@functools.partial(
    pl.kernel,
    out_type=jax.ShapeDtypeStruct((NC, N_PAD, HALF), jnp.float32),
    mesh=_mesh,
    scratch_types=(
        [pltpu.VMEM((CHUNKS2, CHUNK), jnp.int32)] * 2
        + [pltpu.VMEM((CHUNK, HALF), jnp.float32)] * (NBUF + 1)
        + [pltpu.VMEM_SHARED((N_PAD, HALF), jnp.float32)]
        + [pltpu.SemaphoreType.DMA] * (2 * NBUF)
    ),
    compiler_params=_sc_params,
)
def _sc_aggregate(feat_hbm, idx_hbm, out_hbm, *refs):
    sidx_v, didx_v = refs[0], refs[1]
    bufs = refs[2:2 + NBUF]
    zrows = refs[2 + NBUF]
    agg_sh = refs[3 + NBUF]
    gsem = refs[4 + NBUF:4 + 2 * NBUF]
    ssem = refs[4 + 2 * NBUF:4 + 3 * NBUF]

    c = lax.axis_index("c")
    s = lax.axis_index("s")

    pltpu.sync_copy(idx_hbm.at[0, pl.ds(s * CHUNKS2, CHUNKS2)], sidx_v)
    pltpu.sync_copy(idx_hbm.at[1, pl.ds(s * CHUNKS2, CHUNKS2)], didx_v)

    zero16 = jnp.zeros((16,), jnp.float32)

    @pl.loop(0, CHUNK)
    def _(r):
        for k in range(HALF // 16):
            zrows[r, pl.ds(k * 16, 16)] = zero16

    @pl.loop(0, ROWS_PER_TILE // CHUNK)
    def _(j):
        pltpu.sync_copy(
            zrows, agg_sh.at[pl.ds(s * ROWS_PER_TILE + j * CHUNK, CHUNK)])

    plsc.subcore_barrier()

    fsrc = feat_hbm.at[c]

    for j in range(NBUF):
        pltpu.async_copy(fsrc.at[sidx_v.at[j]], bufs[j], gsem[j])

    @pl.loop(0, CHUNKS2 - NBUF, step=NBUF)
    def _(ch):
        for j in range(NBUF):
            pltpu.make_async_copy(
                fsrc.at[sidx_v.at[ch + j]], bufs[j], gsem[j]).wait()
            pltpu.async_copy(bufs[j], agg_sh.at[didx_v.at[ch + j]],
                             ssem[j], add=True)
        for j in range(NBUF):
            pltpu.make_async_copy(
                bufs[j], agg_sh.at[didx_v.at[ch + j]], ssem[j]).wait()
            pltpu.async_copy(
                fsrc.at[sidx_v.at[ch + NBUF + j]], bufs[j], gsem[j])

    for j in range(NBUF):
        ch = CHUNKS2 - NBUF + j
        pltpu.make_async_copy(
            fsrc.at[sidx_v.at[ch]], bufs[j], gsem[j]).wait()
        pltpu.async_copy(bufs[j], agg_sh.at[didx_v.at[ch]],
                         ssem[j], add=True)
    for j in range(NBUF):
        ch = CHUNKS2 - NBUF + j
        pltpu.make_async_copy(
            bufs[j], agg_sh.at[didx_v.at[ch]], ssem[j]).wait()

    plsc.subcore_barrier()
    pltpu.sync_copy(
        agg_sh.at[pl.ds(s * ROWS_PER_TILE, ROWS_PER_TILE)],
        out_hbm.at[c, pl.ds(s * ROWS_PER_TILE, ROWS_PER_TILE)])


def _tc_feat_body(h_ref, od_ref, o_ref):
    od = jnp.sum(od_ref[...], axis=0)
    norm = lax.rsqrt(jnp.maximum(od, 1.0))
    f = h_ref[...] * norm[:, None]
    o_ref[0] = f[:, :HALF]
    o_ref[1] = f[:, HALF:]


def _tc_feat(h, od_p):
    blk = 1024
    return pl.pallas_call(
        _tc_feat_body,
        grid=(N_PAD // blk,),
        in_specs=[
            pl.BlockSpec((blk, DIM), lambda i: (i, 0)),
            pl.BlockSpec((NW, blk), lambda i: (0, i)),
        ],
        out_specs=pl.BlockSpec((NC, blk, HALF), lambda i: (0, i, 0)),
        out_shape=jax.ShapeDtypeStruct((NC, N_PAD, HALF), jnp.float32),
    )(h, od_p)


def _tc_out_body(agg_ref, id_ref, w_ref, b_ref, o_ref):
    ideg = jnp.sum(id_ref[...], axis=0)
    norm = lax.rsqrt(jnp.maximum(ideg, 1.0))
    a0 = agg_ref[0] * norm[:, None]
    a1 = agg_ref[1] * norm[:, None]
    w = w_ref[...]
    x = (jnp.dot(a0, w[:HALF, :], preferred_element_type=jnp.float32)
         + jnp.dot(a1, w[HALF:, :], preferred_element_type=jnp.float32)
         + b_ref[...])
    m = jnp.max(x, axis=1, keepdims=True)
    sh = x - m
    lse = jnp.log(jnp.sum(jnp.exp(sh), axis=1, keepdims=True))
    o_ref[...] = sh - lse


def _tc_out(agg_p, id_p, W, b2):
    blk = 1024
    return pl.pallas_call(
        _tc_out_body,
        grid=(pl.cdiv(N_NODES, blk),),
        in_specs=[
            pl.BlockSpec((NC, blk, HALF), lambda i: (0, i, 0)),
            pl.BlockSpec((NW, blk), lambda i: (0, i)),
            pl.BlockSpec((DIM, DIM), lambda i: (0, 0)),
            pl.BlockSpec((1, DIM), lambda i: (0, 0)),
        ],
        out_specs=pl.BlockSpec((blk, DIM), lambda i: (i, 0)),
        out_shape=jax.ShapeDtypeStruct((N_NODES, DIM), jnp.float32),
    )(agg_p, id_p, W, b2)


def kernel(h, edge_index, W, b):
    ei3 = edge_index.astype(jnp.int32).reshape(2, N_EDGES // CHUNK, CHUNK)
    pad = N_NODES + (jnp.arange(NW * PAD_E, dtype=jnp.int32) % 16)
    padc = jnp.broadcast_to(
        pad.reshape(1, NW * PAD_E // CHUNK, CHUNK),
        (2, NW * PAD_E // CHUNK, CHUNK))
    idx3 = jnp.concatenate([ei3, padc], axis=1)

    od_p, id_p = _sc_degrees(idx3)

    feat = _tc_feat(h, od_p)

    agg_p = _sc_aggregate(feat, idx3)

    return _tc_out(agg_p, id_p, W, b.reshape(1, DIM))
